# Initial kernel scaffold; baseline (speedup 1.0000x reference)
import jax
import jax.numpy as jnp
from jax.experimental import pallas as pl


def kernel(x, noise, Wg, W1, b1, W2, b2):
    raise NotImplementedError("write your pallas kernel here")


# speedup vs baseline: 2.1356x; 2.1356x over previous
import jax
import jax.numpy as jnp
from jax.experimental import pallas as pl
from jax.experimental.pallas import tpu as pltpu

NUM_EXPERTS = 8
BT_TOK = 256


def _gate_body(x_ref, noise_ref, wg_ref, w_ref):
    noisy = x_ref[...] + noise_ref[...]
    logits = jax.lax.dot_general(
        noisy.astype(jnp.bfloat16), wg_ref[...].astype(jnp.bfloat16),
        (((1,), (1,)), ((), ())),
        preferred_element_type=jnp.float32)
    iota = jax.lax.broadcasted_iota(jnp.int32, logits.shape, 1)
    v0 = jnp.max(logits, axis=1, keepdims=True)
    i0 = jnp.min(jnp.where(logits == v0, iota, NUM_EXPERTS), axis=1,
                 keepdims=True)
    sel0 = iota == i0
    l1 = jnp.where(sel0, -jnp.inf, logits)
    v1 = jnp.max(l1, axis=1, keepdims=True)
    i1 = jnp.min(jnp.where(l1 == v1, iota, NUM_EXPERTS), axis=1,
                 keepdims=True)
    sel1 = iota == i1
    w0 = 1.0 / (1.0 + jnp.exp(v1 - v0))
    w_ref[...] = jnp.where(sel0, w0, 0.0) + jnp.where(sel1, 1.0 - w0, 0.0)


def _ffn_body(x_ref, w1_ref, b1_ref, w2_ref, b2_ref, wmat_ref, out_ref):
    e = pl.program_id(1)
    xb = x_ref[...].astype(jnp.bfloat16)
    h = jax.lax.dot_general(
        xb, w1_ref[0], (((1,), (1,)), ((), ())),
        preferred_element_type=jnp.float32)
    h = h + b1_ref[0]
    h = 0.5 * h * (1.0 + jax.lax.erf(h * 0.7071067811865476))
    h2 = jax.lax.dot_general(
        h.astype(jnp.bfloat16), w2_ref[0], (((1,), (1,)), ((), ())),
        preferred_element_type=jnp.float32)
    h2 = h2 + b2_ref[0]
    eiota = jax.lax.broadcasted_iota(jnp.int32, wmat_ref.shape, 1)
    wcol = jnp.sum(jnp.where(eiota == e, wmat_ref[...], 0.0), axis=1,
                   keepdims=True)
    contrib = h2 * wcol

    @pl.when(e == 0)
    def _():
        out_ref[...] = contrib

    @pl.when(e > 0)
    def _():
        out_ref[...] = out_ref[...] + contrib


def kernel(x, noise, Wg, W1, b1, W2, b2):
    b, t, c = x.shape
    e, d_ff, _ = W1.shape
    xf = x.reshape(t, c)

    wmat = pl.pallas_call(
        _gate_body,
        out_shape=jax.ShapeDtypeStruct((t, e), jnp.float32),
    )(xf, noise, Wg)

    n_tb = t // BT_TOK
    out = pl.pallas_call(
        _ffn_body,
        grid=(n_tb, e),
        in_specs=[
            pl.BlockSpec((BT_TOK, c), lambda ti, ei: (ti, 0)),
            pl.BlockSpec((1, d_ff, c), lambda ti, ei: (ei, 0, 0)),
            pl.BlockSpec((1, 1, d_ff), lambda ti, ei: (ei, 0, 0)),
            pl.BlockSpec((1, c, d_ff), lambda ti, ei: (ei, 0, 0)),
            pl.BlockSpec((1, 1, c), lambda ti, ei: (ei, 0, 0)),
            pl.BlockSpec((BT_TOK, e), lambda ti, ei: (ti, 0)),
        ],
        out_specs=pl.BlockSpec((BT_TOK, c), lambda ti, ei: (ti, 0)),
        out_shape=jax.ShapeDtypeStruct((t, c), jnp.float32),
    )(xf, W1.astype(jnp.bfloat16), b1.reshape(e, 1, d_ff),
      W2.astype(jnp.bfloat16), b2.reshape(e, 1, c), wmat)

    return out.reshape(b, t, c).astype(x.dtype)


# speedup vs baseline: 3.9509x; 1.8500x over previous
import functools

import jax
import jax.numpy as jnp
from jax import lax
from jax.experimental import pallas as pl
from jax.experimental.pallas import tpu as pltpu
from jax.experimental.pallas import tpu_sc as plsc

E = 8
K = 2
T = 2048
C = 768
D_FF = 4 * C
BT = 256
NSLOT = K * T
NB = NSLOT // BT + E
CAP_ROWS = NB * BT
NW = 32
SPW = NSLOT // NW
TPW = T // NW
CH = 32


def _route_body(x_ref, noise_ref, wg_ref, w0_ref, w1_ref, pos_ref, meta_ref):
    noisy = x_ref[...] + noise_ref[...]
    logits = lax.dot_general(
        noisy.astype(jnp.bfloat16), wg_ref[...].astype(jnp.bfloat16),
        (((1,), (1,)), ((), ())), preferred_element_type=jnp.float32)
    eio = lax.broadcasted_iota(jnp.int32, logits.shape, 1)
    v0 = jnp.max(logits, axis=1, keepdims=True)
    i0 = jnp.min(jnp.where(logits == v0, eio, E), axis=1, keepdims=True)
    sel0 = eio == i0
    l1 = jnp.where(sel0, -jnp.inf, logits)
    v1 = jnp.max(l1, axis=1, keepdims=True)
    i1 = jnp.min(jnp.where(l1 == v1, eio, E), axis=1, keepdims=True)
    sel1 = eio == i1
    w0 = 1.0 / (1.0 + jnp.exp(v1 - v0))
    w0_ref[...] = w0
    w1_ref[...] = 1.0 - w0

    oh = jnp.concatenate([sel0, sel1], axis=0).astype(jnp.float32)
    tri = (lax.broadcasted_iota(jnp.int32, (128, 128), 1)
           <= lax.broadcasted_iota(jnp.int32, (128, 128), 0)).astype(
               jnp.float32)
    carry = jnp.zeros((1, E), jnp.float32)
    cs_parts = []
    for b in range(NSLOT // 128):
        blk = oh[b * 128:(b + 1) * 128, :]
        cs_b = lax.dot_general(tri, blk, (((1,), (0,)), ((), ())),
                               preferred_element_type=jnp.float32) + carry
        cs_parts.append(cs_b)
        carry = cs_b[127:128, :]
    cs = jnp.concatenate(cs_parts, axis=0)
    counts = carry
    cap = jnp.ceil(counts * (1.0 / BT)) * BT
    upper = (lax.broadcasted_iota(jnp.int32, (E, E), 0)
             < lax.broadcasted_iota(jnp.int32, (E, E), 1)).astype(jnp.float32)
    offs = lax.dot_general(cap, upper, (((1,), (0,)), ((), ())),
                           preferred_element_type=jnp.float32,
                           precision=lax.Precision.HIGHEST)
    sel = jnp.concatenate([sel0, sel1], axis=0)
    posf = jnp.sum(jnp.where(sel, offs + cs - 1.0, 0.0), axis=1,
                   keepdims=True)
    pos_ref[...] = posf.astype(jnp.int32)

    endb = (offs + cap) * (1.0 / BT)
    eio8 = lax.broadcasted_iota(jnp.int32, (E, E), 0)
    ejo8 = lax.broadcasted_iota(jnp.int32, (E, E), 1)
    endb_col = jnp.sum(jnp.where(eio8 == ejo8, endb + jnp.zeros((E, E)), 0.0),
                       axis=1, keepdims=True)
    bidx = lax.broadcasted_iota(jnp.int32, (1, 128), 1).astype(jnp.float32)
    nfull = jnp.sum((endb_col <= bidx).astype(jnp.float32), axis=0,
                    keepdims=True)
    bexp = jnp.minimum(nfull, float(E - 1))
    total = jnp.sum(cap, axis=1, keepdims=True) * (1.0 / BT)
    bval = (bidx < total).astype(jnp.float32)
    meta_ref[...] = jnp.concatenate([bexp, bval], axis=0).astype(jnp.int32)


def _dispatch_body(x_hbm, pos_hbm, xs_hbm, idx_v, rows_v, sem):
    wid = lax.axis_index("s") * 2 + lax.axis_index("c")
    base = wid * SPW
    tbase = lax.rem(base, T)
    pltpu.sync_copy(pos_hbm.at[pl.ds(base, SPW)], idx_v)
    pltpu.sync_copy(x_hbm.at[pl.ds(tbase, SPW)], rows_v)
    pltpu.async_copy(rows_v, xs_hbm.at[idx_v], sem).wait()


def _gffn_body(bexp_ref, bval_ref, xs_ref, w1_ref, b1_ref, w2_ref, b2_ref,
               out_ref):
    b = pl.program_id(0)

    @pl.when(bval_ref[b] != 0)
    def _():
        xb = xs_ref[...].astype(jnp.bfloat16)
        h = lax.dot_general(xb, w1_ref[0].astype(jnp.bfloat16),
                            (((1,), (1,)), ((), ())),
                            preferred_element_type=jnp.float32)
        h = h + b1_ref[0]
        h = 0.5 * h * (1.0 + lax.erf(h * 0.7071067811865476))
        h2 = lax.dot_general(h.astype(jnp.bfloat16),
                             w2_ref[0].astype(jnp.bfloat16),
                             (((1,), (1,)), ((), ())),
                             preferred_element_type=jnp.float32)
        out_ref[...] = h2 + b2_ref[0]


def _combine_body(ys_hbm, pos0_hbm, pos1_hbm, y0_hbm, y1_hbm,
                  idx0_v, idx1_v, rows0_v, rows1_v, sem0, sem1):
    wid = lax.axis_index("s") * 2 + lax.axis_index("c")
    tb = wid * TPW
    pltpu.sync_copy(pos0_hbm.at[pl.ds(tb, TPW)], idx0_v)
    pltpu.sync_copy(pos1_hbm.at[pl.ds(tb, TPW)], idx1_v)
    cp0 = pltpu.async_copy(ys_hbm.at[idx0_v], rows0_v, sem0)
    cp1 = pltpu.async_copy(ys_hbm.at[idx1_v], rows1_v, sem1)
    cp0.wait()
    cp1.wait()
    pltpu.sync_copy(rows0_v, y0_hbm.at[pl.ds(tb, TPW)])
    pltpu.sync_copy(rows1_v, y1_hbm.at[pl.ds(tb, TPW)])


def _mix_body(y0_ref, y1_ref, w0_ref, w1_ref, out_ref):
    out_ref[...] = (y0_ref[...] * w0_ref[...] +
                    y1_ref[...] * w1_ref[...])


def kernel(x, noise, Wg, W1, b1, W2, b2):
    bsz, t, c = x.shape
    xf = x.reshape(t, c)

    w0, w1, pos, meta = pl.pallas_call(
        _route_body,
        out_shape=[
            jax.ShapeDtypeStruct((T, 1), jnp.float32),
            jax.ShapeDtypeStruct((T, 1), jnp.float32),
            jax.ShapeDtypeStruct((NSLOT, 1), jnp.int32),
            jax.ShapeDtypeStruct((2, 128), jnp.int32),
        ],
    )(xf, noise, Wg)

    posflat = pos.reshape(NSLOT)
    mesh = plsc.VectorSubcoreMesh(core_axis_name="c", subcore_axis_name="s",
                                  num_cores=2, num_subcores=16)
    xs = pl.kernel(
        _dispatch_body,
        out_type=jax.ShapeDtypeStruct((CAP_ROWS, C), jnp.float32),
        mesh=mesh,
        scratch_types=[
            pltpu.VMEM((SPW,), jnp.int32),
            pltpu.VMEM((SPW, C), jnp.float32),
            pltpu.SemaphoreType.DMA,
        ],
    )(xf, posflat)

    bexp = meta[0, :NB]
    bval = meta[1, :NB]
    grid_spec = pltpu.PrefetchScalarGridSpec(
        num_scalar_prefetch=2,
        grid=(NB,),
        in_specs=[
            pl.BlockSpec((BT, C), lambda b, be, bv: (b, 0)),
            pl.BlockSpec((1, D_FF, C), lambda b, be, bv: (be[b], 0, 0)),
            pl.BlockSpec((1, 1, D_FF), lambda b, be, bv: (be[b], 0, 0)),
            pl.BlockSpec((1, C, D_FF), lambda b, be, bv: (be[b], 0, 0)),
            pl.BlockSpec((1, 1, C), lambda b, be, bv: (be[b], 0, 0)),
        ],
        out_specs=pl.BlockSpec((BT, C), lambda b, be, bv: (b, 0)),
    )
    ys = pl.pallas_call(
        _gffn_body,
        grid_spec=grid_spec,
        out_shape=jax.ShapeDtypeStruct((CAP_ROWS, C), jnp.float32),
    )(bexp, bval, xs, W1, b1.reshape(E, 1, D_FF), W2, b2.reshape(E, 1, C))

    y0, y1 = pl.kernel(
        _combine_body,
        out_type=[
            jax.ShapeDtypeStruct((T, C), jnp.float32),
            jax.ShapeDtypeStruct((T, C), jnp.float32),
        ],
        mesh=mesh,
        scratch_types=[
            pltpu.VMEM((TPW,), jnp.int32),
            pltpu.VMEM((TPW,), jnp.int32),
            pltpu.VMEM((TPW, C), jnp.float32),
            pltpu.VMEM((TPW, C), jnp.float32),
            pltpu.SemaphoreType.DMA,
            pltpu.SemaphoreType.DMA,
        ],
    )(ys, posflat[:T], posflat[T:])

    out = pl.pallas_call(
        _mix_body,
        grid=(T // BT,),
        in_specs=[
            pl.BlockSpec((BT, C), lambda i: (i, 0)),
            pl.BlockSpec((BT, C), lambda i: (i, 0)),
            pl.BlockSpec((BT, 1), lambda i: (i, 0)),
            pl.BlockSpec((BT, 1), lambda i: (i, 0)),
        ],
        out_specs=pl.BlockSpec((BT, C), lambda i: (i, 0)),
        out_shape=jax.ShapeDtypeStruct((T, C), jnp.float32),
    )(y0, y1, w0, w1)

    return out.reshape(bsz, t, c).astype(x.dtype)


# speedup vs baseline: 4.1390x; 1.0476x over previous
import functools

import jax
import jax.numpy as jnp
from jax import lax
from jax.experimental import pallas as pl
from jax.experimental.pallas import tpu as pltpu
from jax.experimental.pallas import tpu_sc as plsc

E = 8
K = 2
T = 2048
C = 768
D_FF = 4 * C
BT = 256
NSLOT = K * T
NB = NSLOT // BT + E
CAP_ROWS = NB * BT
NW = 32
SPW = NSLOT // NW
TPW = T // NW


def _route_body(x_ref, noise_ref, wg_ref, xbf_ref, wwide_ref, pos_ref,
                meta_ref):
    xa = x_ref[...][:, :C // 2].astype(jnp.bfloat16).astype(jnp.float32)
    xb_ = x_ref[...][:, C // 2:].astype(jnp.bfloat16).astype(jnp.float32)
    ua = lax.shift_right_logical(
        lax.bitcast_convert_type(xa, jnp.uint32), jnp.uint32(16))
    ub = lax.bitcast_convert_type(xb_, jnp.uint32) & jnp.uint32(0xFFFF0000)
    xbf_ref[...] = lax.bitcast_convert_type(ua | ub, jnp.float32)
    noisy = x_ref[...] + noise_ref[...]
    logits = lax.dot_general(
        noisy.astype(jnp.bfloat16), wg_ref[...].astype(jnp.bfloat16),
        (((1,), (1,)), ((), ())), preferred_element_type=jnp.float32)
    eio = lax.broadcasted_iota(jnp.int32, logits.shape, 1)
    v0 = jnp.max(logits, axis=1, keepdims=True)
    i0 = jnp.min(jnp.where(logits == v0, eio, E), axis=1, keepdims=True)
    sel0 = eio == i0
    l1 = jnp.where(sel0, -jnp.inf, logits)
    v1 = jnp.max(l1, axis=1, keepdims=True)
    i1 = jnp.min(jnp.where(l1 == v1, eio, E), axis=1, keepdims=True)
    sel1 = eio == i1
    w0 = 1.0 / (1.0 + jnp.exp(v1 - v0))
    ones = jnp.ones((1, 128), jnp.float32)
    wwide_ref[...] = jnp.concatenate([w0 * ones, (1.0 - w0) * ones], axis=0)

    oh = jnp.concatenate([sel0, sel1], axis=0).astype(jnp.float32)
    tri = (lax.broadcasted_iota(jnp.int32, (128, 128), 1)
           <= lax.broadcasted_iota(jnp.int32, (128, 128), 0)).astype(
               jnp.float32)
    carry = jnp.zeros((1, E), jnp.float32)
    cs_parts = []
    for b in range(NSLOT // 128):
        blk = oh[b * 128:(b + 1) * 128, :]
        cs_b = lax.dot_general(tri, blk, (((1,), (0,)), ((), ())),
                               preferred_element_type=jnp.float32) + carry
        cs_parts.append(cs_b)
        carry = cs_b[127:128, :]
    cs = jnp.concatenate(cs_parts, axis=0)
    counts = carry
    cap = jnp.ceil(counts * (1.0 / BT)) * BT
    upper = (lax.broadcasted_iota(jnp.int32, (E, E), 0)
             < lax.broadcasted_iota(jnp.int32, (E, E), 1)).astype(jnp.float32)
    offs = lax.dot_general(cap, upper, (((1,), (0,)), ((), ())),
                           preferred_element_type=jnp.float32,
                           precision=lax.Precision.HIGHEST)
    sel = jnp.concatenate([sel0, sel1], axis=0)
    posf = jnp.sum(jnp.where(sel, offs + cs - 1.0, 0.0), axis=1,
                   keepdims=True)
    pos_ref[...] = posf.astype(jnp.int32)

    endb = (offs + cap) * (1.0 / BT)
    eio8 = lax.broadcasted_iota(jnp.int32, (E, E), 0)
    ejo8 = lax.broadcasted_iota(jnp.int32, (E, E), 1)
    endb_col = jnp.sum(jnp.where(eio8 == ejo8, endb + jnp.zeros((E, E)), 0.0),
                       axis=1, keepdims=True)
    bidx = lax.broadcasted_iota(jnp.int32, (1, 128), 1).astype(jnp.float32)
    nfull = jnp.sum((endb_col <= bidx).astype(jnp.float32), axis=0,
                    keepdims=True)
    bexp = jnp.minimum(nfull, float(E - 1))
    total = jnp.sum(cap, axis=1, keepdims=True) * (1.0 / BT)
    bval = (bidx < total).astype(jnp.float32)
    meta_ref[...] = jnp.concatenate([bexp, bval], axis=0).astype(jnp.int32)


def _dispatch_body(x_hbm, pos_hbm, wwide_hbm, xs_hbm, wsort_hbm,
                   idx_v, idx2_v, rows_v, wrow_v, sem, sem2):
    wid = lax.axis_index("s") * 2 + lax.axis_index("c")
    base = wid * SPW
    tbase = lax.rem(base, T)
    pltpu.sync_copy(pos_hbm.at[pl.ds(base, SPW)], idx_v)
    pltpu.sync_copy(pos_hbm.at[pl.ds(base, SPW)], idx2_v)
    pltpu.sync_copy(x_hbm.at[pl.ds(tbase, SPW)], rows_v)
    cpx = pltpu.async_copy(rows_v, xs_hbm.at[idx_v], sem)
    pltpu.sync_copy(wwide_hbm.at[pl.ds(base, SPW)], wrow_v)
    pltpu.async_copy(wrow_v, wsort_hbm.at[idx2_v], sem2).wait()
    cpx.wait()


def _gffn_body(bexp_ref, bval_ref, xs_ref, wsort_ref, w1_ref, b1_ref, w2_ref,
               b2_ref, out_ref):
    b = pl.program_id(0)

    @pl.when(bval_ref[b] != 0)
    def _():
        u = lax.bitcast_convert_type(xs_ref[...], jnp.uint32)
        lo = lax.bitcast_convert_type(
            lax.shift_left(u, jnp.uint32(16)), jnp.float32)
        hi = lax.bitcast_convert_type(u & jnp.uint32(0xFFFF0000), jnp.float32)
        xb = jnp.concatenate([lo, hi], axis=1).astype(jnp.bfloat16)
        h = lax.dot_general(xb, w1_ref[0].astype(jnp.bfloat16),
                            (((1,), (1,)), ((), ())),
                            preferred_element_type=jnp.float32)
        h = h + b1_ref[0]
        h = 0.5 * h * (1.0 + lax.erf(h * 0.7071067811865476))
        h2 = lax.dot_general(h.astype(jnp.bfloat16),
                             w2_ref[0].astype(jnp.bfloat16),
                             (((1,), (1,)), ((), ())),
                             preferred_element_type=jnp.float32)
        res = (h2 + b2_ref[0]) * wsort_ref[:, 0:1]
        ra = res[:, :C // 2].astype(jnp.bfloat16).astype(jnp.float32)
        rb = res[:, C // 2:].astype(jnp.bfloat16).astype(jnp.float32)
        ua = lax.shift_right_logical(
            lax.bitcast_convert_type(ra, jnp.uint32), jnp.uint32(16))
        ub = lax.bitcast_convert_type(rb, jnp.uint32) & jnp.uint32(0xFFFF0000)
        out_ref[...] = lax.bitcast_convert_type(ua | ub, jnp.float32)


def _combine_body(ys_hbm, pos0_hbm, pos1_hbm, out_hbm,
                  idx0_v, idx1_v, rows0_v, rows1_v, ob_v, sem0, sem1):
    wid = lax.axis_index("s") * 2 + lax.axis_index("c")
    tb = wid * TPW
    pltpu.sync_copy(pos0_hbm.at[pl.ds(tb, TPW)], idx0_v)
    pltpu.sync_copy(pos1_hbm.at[pl.ds(tb, TPW)], idx1_v)
    cp0 = pltpu.async_copy(ys_hbm.at[idx0_v], rows0_v, sem0)
    cp1 = pltpu.async_copy(ys_hbm.at[idx1_v], rows1_v, sem1)
    cp0.wait()
    cp1.wait()
    mask = jnp.int32(-65536)
    sh = jnp.int32(16)

    def body(j, _):
        bc = lax.bitcast_convert_type
        for v in range(C // 32):
            sl = pl.ds(v * 16, 16)
            u0 = bc(rows0_v[j, sl], jnp.int32)
            u1 = bc(rows1_v[j, sl], jnp.int32)
            lo = (bc(lax.shift_left(u0, sh), jnp.float32) +
                  bc(lax.shift_left(u1, sh), jnp.float32))
            hi = (bc(u0 & mask, jnp.float32) +
                  bc(u1 & mask, jnp.float32))
            ob_v[j, sl] = lo
            ob_v[j, pl.ds(C // 2 + v * 16, 16)] = hi
        return 0

    jax.lax.fori_loop(0, TPW, body, 0, unroll=False)
    pltpu.sync_copy(ob_v, out_hbm.at[pl.ds(tb, TPW)])


def kernel(x, noise, Wg, W1, b1, W2, b2):
    bsz, t, c = x.shape
    xf = x.reshape(t, c)

    xbf, wwide, pos, meta = pl.pallas_call(
        _route_body,
        out_shape=[
            jax.ShapeDtypeStruct((T, C // 2), jnp.float32),
            jax.ShapeDtypeStruct((NSLOT, 128), jnp.float32),
            jax.ShapeDtypeStruct((NSLOT, 1), jnp.int32),
            jax.ShapeDtypeStruct((2, 128), jnp.int32),
        ],
    )(xf, noise, Wg)

    posflat = pos.reshape(NSLOT)
    mesh = plsc.VectorSubcoreMesh(core_axis_name="c", subcore_axis_name="s",
                                  num_cores=2, num_subcores=16)
    xs, wsort = pl.kernel(
        _dispatch_body,
        out_type=[
            jax.ShapeDtypeStruct((CAP_ROWS, C // 2), jnp.float32),
            jax.ShapeDtypeStruct((CAP_ROWS, 128), jnp.float32),
        ],
        mesh=mesh,
        scratch_types=[
            pltpu.VMEM((SPW,), jnp.int32),
            pltpu.VMEM((SPW,), jnp.int32),
            pltpu.VMEM((SPW, C // 2), jnp.float32),
            pltpu.VMEM((SPW, 128), jnp.float32),
            pltpu.SemaphoreType.DMA,
            pltpu.SemaphoreType.DMA,
        ],
    )(xbf, posflat, wwide)

    bexp = meta[0, :NB]
    bval = meta[1, :NB]
    grid_spec = pltpu.PrefetchScalarGridSpec(
        num_scalar_prefetch=2,
        grid=(NB,),
        in_specs=[
            pl.BlockSpec((BT, C // 2), lambda b, be, bv: (b, 0)),
            pl.BlockSpec((BT, 128), lambda b, be, bv: (b, 0)),
            pl.BlockSpec((1, D_FF, C), lambda b, be, bv: (be[b], 0, 0)),
            pl.BlockSpec((1, 1, D_FF), lambda b, be, bv: (be[b], 0, 0)),
            pl.BlockSpec((1, C, D_FF), lambda b, be, bv: (be[b], 0, 0)),
            pl.BlockSpec((1, 1, C), lambda b, be, bv: (be[b], 0, 0)),
        ],
        out_specs=pl.BlockSpec((BT, C // 2), lambda b, be, bv: (b, 0)),
    )
    ys = pl.pallas_call(
        _gffn_body,
        grid_spec=grid_spec,
        out_shape=jax.ShapeDtypeStruct((CAP_ROWS, C // 2), jnp.float32),
    )(bexp, bval, xs, wsort, W1, b1.reshape(E, 1, D_FF), W2,
      b2.reshape(E, 1, C))

    out = pl.kernel(
        _combine_body,
        out_type=jax.ShapeDtypeStruct((T, C), jnp.float32),
        mesh=mesh,
        scratch_types=[
            pltpu.VMEM((TPW,), jnp.int32),
            pltpu.VMEM((TPW,), jnp.int32),
            pltpu.VMEM((TPW, C // 2), jnp.float32),
            pltpu.VMEM((TPW, C // 2), jnp.float32),
            pltpu.VMEM((TPW, C), jnp.float32),
            pltpu.SemaphoreType.DMA,
            pltpu.SemaphoreType.DMA,
        ],
    )(ys, posflat[:T], posflat[T:])

    return out.reshape(bsz, t, c).astype(x.dtype)
